# baseline (speedup 1.0000x reference)
import functools

import jax
import jax.numpy as jnp
from jax import lax
from jax.experimental import pallas as pl
from jax.experimental.pallas import tpu as pltpu
from jax.experimental.pallas import tpu_sc as plsc

B = 16384
D = 16
NC = 2
NS = 16
NW = NC * NS
BPW = B // NW
CHUNK = 128
NCH = BPW // CHUNK


def _sc_partials(x, xn, yn, human_table, virus_table):
    mesh = plsc.VectorSubcoreMesh(core_axis_name="c", subcore_axis_name="s")

    @functools.partial(
        pl.kernel,
        mesh=mesh,
        compiler_params=pltpu.CompilerParams(
            needs_layout_passes=False, use_tc_tiling_on_sc=False),
        out_type=jax.ShapeDtypeStruct((NW, D), jnp.float32),
        scratch_types=[
            pltpu.VMEM((NCH, CHUNK), jnp.int32),
            pltpu.VMEM((NCH, CHUNK), jnp.int32),
            pltpu.VMEM((BPW, D), jnp.float32),
            pltpu.VMEM((BPW, D), jnp.float32),
            pltpu.VMEM((BPW, D), jnp.float32),
            pltpu.VMEM((D,), jnp.float32),
            pltpu.SemaphoreType.DMA,
            pltpu.SemaphoreType.DMA,
        ],
    )
    def k(x_hbm, xn_hbm, yn_hbm, ht_hbm, vt_hbm, out_hbm,
          idx_u, idx_v, rows_u, rows_v, x_v, acc_v, sem_u, sem_v):
        wid = lax.axis_index("s") * NC + lax.axis_index("c")
        base = wid * BPW

        for c in range(NCH):
            pltpu.sync_copy(xn_hbm.at[pl.ds(base + c * CHUNK, CHUNK)], idx_u.at[jnp.int32(c)])
            pltpu.sync_copy(yn_hbm.at[pl.ds(base + c * CHUNK, CHUNK)], idx_v.at[jnp.int32(c)])
        for c in range(NCH):
            pltpu.async_copy(ht_hbm.at[idx_u.at[jnp.int32(c)]],
                             rows_u.at[pl.ds(c * CHUNK, CHUNK)], sem_u)
            pltpu.async_copy(vt_hbm.at[idx_v.at[jnp.int32(c)]],
                             rows_v.at[pl.ds(c * CHUNK, CHUNK)], sem_v)
        pltpu.sync_copy(x_hbm.at[pl.ds(base, BPW)], x_v)
        for c in range(NCH):
            pltpu.make_async_copy(ht_hbm.at[idx_u.at[jnp.int32(c)]],
                                  rows_u.at[pl.ds(c * CHUNK, CHUNK)], sem_u).wait()
            pltpu.make_async_copy(vt_hbm.at[idx_v.at[jnp.int32(c)]],
                                  rows_v.at[pl.ds(c * CHUNK, CHUNK)], sem_v).wait()

        def body(i, acc):
            s = jnp.sum(rows_u[i] * x_v[i])
            return acc + s * rows_v[i]

        acc = lax.fori_loop(0, BPW, body, jnp.zeros((D,), jnp.float32))
        acc_v[...] = acc
        pltpu.sync_copy(acc_v, out_hbm.at[wid])

    return k(x, xn, yn, human_table, virus_table)


def _tc_finish_body(y_ref, p_ref, o_ref):
    t = jnp.sum(p_ref[...], axis=0)
    o_ref[...] = jnp.sum(y_ref[...] * t[None, :], axis=1)


def _tc_finish(y, partials):
    return pl.pallas_call(
        _tc_finish_body,
        out_shape=jax.ShapeDtypeStruct((B,), jnp.float32),
    )(y, partials)


def kernel(x, y, x_nodes, y_nodes, human_table, virus_table):
    xn = x_nodes.astype(jnp.int32)
    yn = y_nodes.astype(jnp.int32)
    partials = _sc_partials(x, xn, yn, human_table, virus_table)
    out = _tc_finish(y, partials)
    return out.astype(jnp.float64)
